# initial kernel scaffold (unmeasured)
import jax
import jax.numpy as jnp
from jax import lax
from jax.experimental import pallas as pl
from jax.experimental.pallas import tpu as pltpu


def kernel(
    x,
):
    def body(*refs):
        pass

    out_shape = jax.ShapeDtypeStruct(..., jnp.float32)
    return pl.pallas_call(body, out_shape=out_shape)(...)


# baseline (device time: 18893 ns/iter reference)
import jax
import jax.numpy as jnp
from jax import lax
from jax.experimental import pallas as pl
from jax.experimental.pallas import tpu as pltpu


def kernel(x):
    _, m, n = x.shape
    half = n // 2

    def body(x_ref, out_ref, send_buf, recv_buf, send_sem, recv_sem):
        my_x = lax.axis_index("x")
        my_y = lax.axis_index("y")
        my_z = lax.axis_index("z")
        peer_z = 1 - my_z

        send_buf[...] = x_ref[0, :, pl.ds(peer_z * half, half)].astype(
            jnp.bfloat16
        )

        barrier_sem = pltpu.get_barrier_semaphore()
        pl.semaphore_signal(
            barrier_sem,
            inc=1,
            device_id=(my_x, my_y, peer_z),
            device_id_type=pl.DeviceIdType.MESH,
        )
        pl.semaphore_wait(barrier_sem, 1)

        rdma = pltpu.make_async_remote_copy(
            src_ref=send_buf,
            dst_ref=recv_buf,
            send_sem=send_sem,
            recv_sem=recv_sem,
            device_id=(my_x, my_y, peer_z),
            device_id_type=pl.DeviceIdType.MESH,
        )
        rdma.start()
        rdma.wait()
        out_ref[...] = (
            x_ref[0, :, pl.ds(my_z * half, half)]
            + recv_buf[...].astype(jnp.float32)
        )

    return pl.pallas_call(
        body,
        out_shape=jax.ShapeDtypeStruct((m, half), jnp.float32),
        in_specs=[pl.BlockSpec(memory_space=pltpu.VMEM)],
        out_specs=pl.BlockSpec(memory_space=pltpu.VMEM),
        scratch_shapes=[
            pltpu.VMEM((m, half), jnp.bfloat16),
            pltpu.VMEM((m, half), jnp.bfloat16),
            pltpu.SemaphoreType.DMA,
            pltpu.SemaphoreType.DMA,
        ],
        compiler_params=pltpu.CompilerParams(collective_id=0),
    )(x)


# device time: 18746 ns/iter; 1.0078x vs baseline; 1.0078x over previous
import jax
import jax.numpy as jnp
from jax import lax
from jax.experimental import pallas as pl
from jax.experimental.pallas import tpu as pltpu


N_CHUNK = 4


def kernel(x):
    _, m, n = x.shape
    half = n // 2
    rows = m // N_CHUNK

    def body(x_ref, out_ref, send_buf, recv_buf, send_sems, recv_sems):
        my_x = lax.axis_index("x")
        my_y = lax.axis_index("y")
        my_z = lax.axis_index("z")
        peer_z = 1 - my_z

        def stage(k):
            send_buf[pl.ds(k * rows, rows), :] = x_ref[
                0, pl.ds(k * rows, rows), pl.ds(peer_z * half, half)
            ].astype(jnp.bfloat16)

        barrier_sem = pltpu.get_barrier_semaphore()
        pl.semaphore_signal(
            barrier_sem,
            inc=1,
            device_id=(my_x, my_y, peer_z),
            device_id_type=pl.DeviceIdType.MESH,
        )
        stage(0)
        pl.semaphore_wait(barrier_sem, 1)

        rdmas = []
        for k in range(N_CHUNK):
            rdma = pltpu.make_async_remote_copy(
                src_ref=send_buf.at[pl.ds(k * rows, rows)],
                dst_ref=recv_buf.at[pl.ds(k * rows, rows)],
                send_sem=send_sems.at[k],
                recv_sem=recv_sems.at[k],
                device_id=(my_x, my_y, peer_z),
                device_id_type=pl.DeviceIdType.MESH,
            )
            rdma.start()
            rdmas.append(rdma)
            if k + 1 < N_CHUNK:
                stage(k + 1)

        for k in range(N_CHUNK):
            rdmas[k].wait_recv()
            out_ref[pl.ds(k * rows, rows), :] = (
                x_ref[0, pl.ds(k * rows, rows), pl.ds(my_z * half, half)]
                + recv_buf[pl.ds(k * rows, rows), :].astype(jnp.float32)
            )
        for k in range(N_CHUNK):
            rdmas[k].wait_send()

    return pl.pallas_call(
        body,
        out_shape=jax.ShapeDtypeStruct((m, half), jnp.float32),
        in_specs=[pl.BlockSpec(memory_space=pltpu.VMEM)],
        out_specs=pl.BlockSpec(memory_space=pltpu.VMEM),
        scratch_shapes=[
            pltpu.VMEM((m, half), jnp.bfloat16),
            pltpu.VMEM((m, half), jnp.bfloat16),
            pltpu.SemaphoreType.DMA((N_CHUNK,)),
            pltpu.SemaphoreType.DMA((N_CHUNK,)),
        ],
        compiler_params=pltpu.CompilerParams(collective_id=0),
    )(x)


# device time: 17062 ns/iter; 1.1073x vs baseline; 1.0987x over previous
import jax
import jax.numpy as jnp
from jax import lax
from jax.experimental import pallas as pl
from jax.experimental.pallas import tpu as pltpu

N_CHUNK = 4


def kernel(x):
    _, m, n = x.shape
    half = n // 2
    mhalf = m // 2
    rows = mhalf // N_CHUNK

    def body(
        x_ref,
        out_ref,
        zbuf_s,
        zbuf_r,
        xbuf_s,
        xbuf_r,
        zs_sems,
        zr_sems,
        xs_sems,
        xr_sems,
    ):
        my_x = lax.axis_index("x")
        my_y = lax.axis_index("y")
        my_z = lax.axis_index("z")
        peer_z = 1 - my_z
        peer_x = 1 - my_x
        row0 = my_x * mhalf
        orow0 = peer_x * mhalf

        barrier_sem = pltpu.get_barrier_semaphore()
        for dev in ((my_x, my_y, peer_z), (peer_x, my_y, my_z)):
            pl.semaphore_signal(
                barrier_sem,
                inc=1,
                device_id=dev,
                device_id_type=pl.DeviceIdType.MESH,
            )
        zbuf_s[...] = x_ref[
            0, pl.ds(row0, mhalf), pl.ds(peer_z * half, half)
        ].astype(jnp.bfloat16)
        pl.semaphore_wait(barrier_sem, 2)

        z_rdmas = []
        for c in range(N_CHUNK):
            r = pltpu.make_async_remote_copy(
                src_ref=zbuf_s.at[pl.ds(c * rows, rows)],
                dst_ref=zbuf_r.at[pl.ds(c * rows, rows)],
                send_sem=zs_sems.at[c],
                recv_sem=zr_sems.at[c],
                device_id=(my_x, my_y, peer_z),
                device_id_type=pl.DeviceIdType.MESH,
            )
            r.start()
            z_rdmas.append(r)

        x_rdmas = []
        for c in range(N_CHUNK):
            z_rdmas[c].wait_recv()
            loc = x_ref[
                0, pl.ds(row0 + c * rows, rows), pl.ds(my_z * half, half)
            ]
            s = loc + zbuf_r[pl.ds(c * rows, rows), :].astype(jnp.float32)
            out_ref[pl.ds(row0 + c * rows, rows), :] = s
            xbuf_s[pl.ds(c * rows, rows), :] = s.astype(jnp.bfloat16)
            r = pltpu.make_async_remote_copy(
                src_ref=xbuf_s.at[pl.ds(c * rows, rows)],
                dst_ref=xbuf_r.at[pl.ds(c * rows, rows)],
                send_sem=xs_sems.at[c],
                recv_sem=xr_sems.at[c],
                device_id=(peer_x, my_y, my_z),
                device_id_type=pl.DeviceIdType.MESH,
            )
            r.start()
            x_rdmas.append(r)

        for c in range(N_CHUNK):
            x_rdmas[c].wait_recv()
            out_ref[pl.ds(orow0 + c * rows, rows), :] = xbuf_r[
                pl.ds(c * rows, rows), :
            ].astype(jnp.float32)

        for c in range(N_CHUNK):
            z_rdmas[c].wait_send()
            x_rdmas[c].wait_send()

    return pl.pallas_call(
        body,
        out_shape=jax.ShapeDtypeStruct((m, half), jnp.float32),
        in_specs=[pl.BlockSpec(memory_space=pltpu.VMEM)],
        out_specs=pl.BlockSpec(memory_space=pltpu.VMEM),
        scratch_shapes=[
            pltpu.VMEM((mhalf, half), jnp.bfloat16),
            pltpu.VMEM((mhalf, half), jnp.bfloat16),
            pltpu.VMEM((mhalf, half), jnp.bfloat16),
            pltpu.VMEM((mhalf, half), jnp.bfloat16),
            pltpu.SemaphoreType.DMA((N_CHUNK,)),
            pltpu.SemaphoreType.DMA((N_CHUNK,)),
            pltpu.SemaphoreType.DMA((N_CHUNK,)),
            pltpu.SemaphoreType.DMA((N_CHUNK,)),
        ],
        compiler_params=pltpu.CompilerParams(collective_id=0),
    )(x)


# device time: 16498 ns/iter; 1.1452x vs baseline; 1.0342x over previous
import jax
import jax.numpy as jnp
from jax import lax
from jax.experimental import pallas as pl
from jax.experimental.pallas import tpu as pltpu

N_CHUNK = 8


def kernel(x):
    _, m, n = x.shape
    half = n // 2
    mhalf = m // 2
    rows = mhalf // N_CHUNK

    def body(
        x_ref,
        out_ref,
        zbuf_s,
        zbuf_r,
        xbuf_s,
        xbuf_r,
        zs_sems,
        zr_sems,
        xs_sems,
        xr_sems,
    ):
        my_x = lax.axis_index("x")
        my_y = lax.axis_index("y")
        my_z = lax.axis_index("z")
        peer_z = 1 - my_z
        peer_x = 1 - my_x
        row0 = my_x * mhalf
        orow0 = peer_x * mhalf

        barrier_sem = pltpu.get_barrier_semaphore()
        for dev in ((my_x, my_y, peer_z), (peer_x, my_y, my_z)):
            pl.semaphore_signal(
                barrier_sem,
                inc=1,
                device_id=dev,
                device_id_type=pl.DeviceIdType.MESH,
            )
        zbuf_s[...] = x_ref[
            0, pl.ds(row0, mhalf), pl.ds(peer_z * half, half)
        ].astype(jnp.bfloat16)
        pl.semaphore_wait(barrier_sem, 2)

        z_rdmas = []
        for c in range(N_CHUNK):
            r = pltpu.make_async_remote_copy(
                src_ref=zbuf_s.at[pl.ds(c * rows, rows)],
                dst_ref=zbuf_r.at[pl.ds(c * rows, rows)],
                send_sem=zs_sems.at[c],
                recv_sem=zr_sems.at[c],
                device_id=(my_x, my_y, peer_z),
                device_id_type=pl.DeviceIdType.MESH,
            )
            r.start()
            z_rdmas.append(r)

        x_rdmas = []
        for c in range(N_CHUNK):
            z_rdmas[c].wait_recv()
            loc = x_ref[
                0, pl.ds(row0 + c * rows, rows), pl.ds(my_z * half, half)
            ]
            s = loc + zbuf_r[pl.ds(c * rows, rows), :].astype(jnp.float32)
            out_ref[pl.ds(row0 + c * rows, rows), :] = s
            xbuf_s[pl.ds(c * rows, rows), :] = s.astype(jnp.bfloat16)
            r = pltpu.make_async_remote_copy(
                src_ref=xbuf_s.at[pl.ds(c * rows, rows)],
                dst_ref=xbuf_r.at[pl.ds(c * rows, rows)],
                send_sem=xs_sems.at[c],
                recv_sem=xr_sems.at[c],
                device_id=(peer_x, my_y, my_z),
                device_id_type=pl.DeviceIdType.MESH,
            )
            r.start()
            x_rdmas.append(r)

        for c in range(N_CHUNK):
            x_rdmas[c].wait_recv()
            out_ref[pl.ds(orow0 + c * rows, rows), :] = xbuf_r[
                pl.ds(c * rows, rows), :
            ].astype(jnp.float32)

        for c in range(N_CHUNK):
            z_rdmas[c].wait_send()
            x_rdmas[c].wait_send()

    return pl.pallas_call(
        body,
        out_shape=jax.ShapeDtypeStruct((m, half), jnp.float32),
        in_specs=[pl.BlockSpec(memory_space=pltpu.VMEM)],
        out_specs=pl.BlockSpec(memory_space=pltpu.VMEM),
        scratch_shapes=[
            pltpu.VMEM((mhalf, half), jnp.bfloat16),
            pltpu.VMEM((mhalf, half), jnp.bfloat16),
            pltpu.VMEM((mhalf, half), jnp.bfloat16),
            pltpu.VMEM((mhalf, half), jnp.bfloat16),
            pltpu.SemaphoreType.DMA((N_CHUNK,)),
            pltpu.SemaphoreType.DMA((N_CHUNK,)),
            pltpu.SemaphoreType.DMA((N_CHUNK,)),
            pltpu.SemaphoreType.DMA((N_CHUNK,)),
        ],
        compiler_params=pltpu.CompilerParams(collective_id=0),
    )(x)
